# baseline (speedup 1.0000x reference)
import functools

import jax
import jax.numpy as jnp
from jax import lax
from jax.experimental import pallas as pl
from jax.experimental.pallas import tpu as pltpu
from jax.experimental.pallas import tpu_sc as plsc

N_NODES = 10000
NPAD = 10240
E_EDGES = 320000
EPAD = 327680
G_GRAPHS = 64
BLK = 1024
GRID = NPAD // BLK
KCH = 128
EPT = EPAD // 16
NCH = EPT // KCH
RPT = NPAD // 16


@functools.cache
def _make_agg(dh):
    mesh = plsc.VectorSubcoreMesh(core_axis_name="c", subcore_axis_name="s")

    @functools.partial(
        pl.kernel,
        out_type=jax.ShapeDtypeStruct((2 * NPAD, dh), jnp.float32),
        mesh=mesh,
        scratch_types=[
            pltpu.VMEM((KCH,), jnp.int32),
            pltpu.VMEM((KCH,), jnp.int32),
            pltpu.VMEM((KCH, dh), jnp.float32),
            pltpu.VMEM_SHARED((NPAD, dh), jnp.float32),
            pltpu.SemaphoreType.DMA,
        ],
    )
    def agg(h_hbm, src_hbm, dst_hbm, out_hbm, idxs, idxd, rows, accum, sem):
        c = lax.axis_index("c")
        s = lax.axis_index("s")

        def _zrow(i, _):
            for g in range(dh // 16):
                rows[i, pl.ds(g * 16, 16)] = jnp.zeros((16,), jnp.float32)
            return 0

        lax.fori_loop(0, KCH, _zrow, 0)

        def _zacc(k, _):
            pltpu.sync_copy(rows, accum.at[pl.ds(s * RPT + k * KCH, KCH)])
            return 0

        lax.fori_loop(0, RPT // KCH, _zacc, 0)
        plsc.subcore_barrier()

        def _step(i, _):
            eb = s * EPT + i * KCH
            pltpu.sync_copy(src_hbm.at[pl.ds(c * EPAD + eb, KCH)], idxs)
            pltpu.sync_copy(dst_hbm.at[pl.ds(eb, KCH)], idxd)
            pltpu.async_copy(h_hbm.at[idxs], rows, sem).wait()
            pltpu.sync_copy(rows, accum.at[idxd], add=True)
            return 0

        lax.fori_loop(0, NCH, _step, 0)
        plsc.subcore_barrier()

        pltpu.sync_copy(accum.at[pl.ds(s * RPT, RPT)],
                        out_hbm.at[pl.ds(c * NPAD + s * RPT, RPT)])

    return agg


def _rowmask(j):
    r = lax.broadcasted_iota(jnp.int32, (BLK, 1), 0) + j * BLK
    return r < N_NODES


def _k2_body(dh, eps_ref, h0, h1, a0, a1, w1, u_ref, st_ref):
    j = pl.program_id(0)
    e = 1.0 + eps_ref[0, 0]
    x0 = e * h0[...] + a0[...]
    x1 = e * h1[...] + a1[...]
    dn = (((1,), (0,)), ((), ()))
    u = lax.dot_general(x0, w1[:dh, :], dn, preferred_element_type=jnp.float32)
    u = u + lax.dot_general(x1, w1[dh:, :], dn,
                            preferred_element_type=jnp.float32)
    u_ref[...] = u
    um = jnp.where(_rowmask(j), u, 0.0)
    s0 = jnp.sum(um, axis=0, keepdims=True)
    s1 = jnp.sum(um * um, axis=0, keepdims=True)
    upd = jnp.concatenate(
        [s0, s1, jnp.zeros((6, s0.shape[1]), jnp.float32)], axis=0)

    @pl.when(j == 0)
    def _():
        st_ref[...] = upd

    @pl.when(j > 0)
    def _():
        st_ref[...] = st_ref[...] + upd


@functools.cache
def _make_k2(dh):
    fullspec = lambda shp: pl.BlockSpec(shp, lambda j: (0, 0))
    half = lambda off: pl.BlockSpec((BLK, dh), lambda j, o=off: (o + j, 0))
    return pl.pallas_call(
        functools.partial(_k2_body, dh),
        grid=(GRID,),
        in_specs=[
            pl.BlockSpec(memory_space=pltpu.SMEM),
            half(0), half(GRID),
            half(0), half(GRID),
            fullspec((2 * dh, 128)),
        ],
        out_specs=[
            pl.BlockSpec((BLK, 128), lambda j: (j, 0)),
            fullspec((8, 128)),
        ],
        out_shape=[
            jax.ShapeDtypeStruct((NPAD, 128), jnp.float32),
            jax.ShapeDtypeStruct((8, 128), jnp.float32),
        ],
    )


def _bn_coeffs(st_ref, g_ref, b_ref):
    mu = st_ref[0:1, :] / N_NODES
    var = st_ref[1:2, :] / N_NODES - mu * mu
    sc = g_ref[...] * lax.rsqrt(var + 1e-5)
    sh = b_ref[...] - mu * sc
    return sc, sh


def _k3_body(u, st, g1, b1, w2, v_ref, vst_ref):
    j = pl.program_id(0)
    sc, sh = _bn_coeffs(st, g1, b1)
    h2 = jnp.maximum(u[...] * sc + sh, 0.0)
    v = lax.dot_general(h2, w2[...], (((1,), (0,)), ((), ())),
                        preferred_element_type=jnp.float32)
    v_ref[...] = v
    vm = jnp.where(_rowmask(j), v, 0.0)
    s0 = jnp.sum(vm, axis=0, keepdims=True)
    s1 = jnp.sum(vm * vm, axis=0, keepdims=True)
    upd = jnp.concatenate(
        [s0, s1, jnp.zeros((6, s0.shape[1]), jnp.float32)], axis=0)

    @pl.when(j == 0)
    def _():
        vst_ref[...] = upd

    @pl.when(j > 0)
    def _():
        vst_ref[...] = vst_ref[...] + upd


_K3 = None


def _make_k3():
    global _K3
    if _K3 is None:
        fullspec = lambda shp: pl.BlockSpec(shp, lambda j: (0, 0))
        _K3 = pl.pallas_call(
            _k3_body,
            grid=(GRID,),
            in_specs=[
                pl.BlockSpec((BLK, 128), lambda j: (j, 0)),
                fullspec((8, 128)),
                fullspec((1, 128)), fullspec((1, 128)),
                fullspec((128, 256)),
            ],
            out_specs=[
                pl.BlockSpec((BLK, 256), lambda j: (j, 0)),
                fullspec((8, 256)),
            ],
            out_shape=[
                jax.ShapeDtypeStruct((NPAD, 256), jnp.float32),
                jax.ShapeDtypeStruct((8, 256), jnp.float32),
            ],
        )
    return _K3


def _k3f_body(u, st, g1, b1, w2, v_ref):
    sc, sh = _bn_coeffs(st, g1, b1)
    h2 = jnp.maximum(u[...] * sc + sh, 0.0)
    v = lax.dot_general(h2, w2[...], (((1,), (0,)), ((), ())),
                        preferred_element_type=jnp.float32)
    v_ref[...] = jnp.maximum(v, 0.0)


_K3F = None


def _make_k3f():
    global _K3F
    if _K3F is None:
        fullspec = lambda shp: pl.BlockSpec(shp, lambda j: (0, 0))
        _K3F = pl.pallas_call(
            _k3f_body,
            grid=(GRID,),
            in_specs=[
                pl.BlockSpec((BLK, 128), lambda j: (j, 0)),
                fullspec((8, 128)),
                fullspec((1, 128)), fullspec((1, 128)),
                fullspec((128, 128)),
            ],
            out_specs=pl.BlockSpec((BLK, 128), lambda j: (j, 0)),
            out_shape=jax.ShapeDtypeStruct((NPAD, 128), jnp.float32),
        )
    return _K3F


def _k1_body(v, vst, g2, b2, h_ref):
    sc, sh = _bn_coeffs(vst, g2, b2)
    h_ref[...] = jnp.maximum(v[...] * sc + sh, 0.0)


_K1 = None


def _make_k1():
    global _K1
    if _K1 is None:
        _K1 = pl.pallas_call(
            _k1_body,
            grid=(2, GRID),
            in_specs=[
                pl.BlockSpec((BLK, 128), lambda c, j: (j, c)),
                pl.BlockSpec((8, 128), lambda c, j: (0, c)),
                pl.BlockSpec((1, 128), lambda c, j: (0, c)),
                pl.BlockSpec((1, 128), lambda c, j: (0, c)),
            ],
            out_specs=pl.BlockSpec((BLK, 128), lambda c, j: (c * GRID + j, 0)),
            out_shape=jax.ShapeDtypeStruct((2 * NPAD, 128), jnp.float32),
        )
    return _K1


def _pool_body(bcol, h, out_ref, acc, cnt):
    j = pl.program_id(0)
    gids = lax.broadcasted_iota(jnp.int32, (BLK, G_GRAPHS), 1)
    oh = (bcol[...] == gids).astype(jnp.float32)
    dn = (((0,), (0,)), ((), ()))

    @pl.when(j == 0)
    def _():
        acc[...] = jnp.zeros((G_GRAPHS, 128), jnp.float32)
        cnt[...] = jnp.zeros((G_GRAPHS, 128), jnp.float32)

    acc[...] = acc[...] + lax.dot_general(
        oh, h[...], dn, preferred_element_type=jnp.float32)
    cnt[...] = cnt[...] + lax.dot_general(
        oh, jnp.ones((BLK, 128), jnp.float32), dn,
        preferred_element_type=jnp.float32)
    out_ref[...] = acc[...] / jnp.maximum(cnt[...], 1.0)


_POOL = None


def _make_pool():
    global _POOL
    if _POOL is None:
        _POOL = pl.pallas_call(
            _pool_body,
            grid=(GRID,),
            in_specs=[
                pl.BlockSpec((BLK, 1), lambda j: (j, 0)),
                pl.BlockSpec((BLK, 128), lambda j: (j, 0)),
            ],
            out_specs=pl.BlockSpec((G_GRAPHS, 128), lambda j: (0, 0)),
            out_shape=jax.ShapeDtypeStruct((G_GRAPHS, 128), jnp.float32),
            scratch_shapes=[
                pltpu.VMEM((G_GRAPHS, 128), jnp.float32),
                pltpu.VMEM((G_GRAPHS, 128), jnp.float32),
            ],
        )
    return _POOL


def kernel(x, edge_index, edge_attribute, batch, params):
    src = edge_index[0]
    dst = edge_index[1]
    npad = EPAD - E_EDGES
    srcp = jnp.concatenate([src, jnp.zeros((npad,), jnp.int32)])
    dstp = jnp.concatenate([dst, jnp.full((npad,), N_NODES, jnp.int32)])
    src2 = jnp.concatenate([srcp, srcp + NPAD])

    xp = jnp.pad(x, ((0, NPAD - N_NODES), (0, 0)))
    hcat = jnp.concatenate([xp, jnp.zeros((NPAD, 128), jnp.float32)], axis=0)

    h7 = None
    for i, p in enumerate(params):
        agg = _make_agg(128)(hcat, src2, dstp)
        eps2 = (p['eps'].astype(jnp.float32)).reshape(1, 1)
        w1 = p['W1'] if i > 0 else jnp.pad(p['W1'], ((0, 128), (0, 0)))
        u, ust = _make_k2(128)(eps2, hcat, hcat, agg, agg, w1)
        g1 = p['g1'].reshape(1, 128)
        b1 = p['b1'].reshape(1, 128)
        if i < 6:
            v, vst = _make_k3()(u, ust, g1, b1, p['W2'])
            hcat = _make_k1()(v, vst, p['g2'].reshape(1, 256),
                              p['b2'].reshape(1, 256))
        else:
            w2p = jnp.pad(p['W2'], ((0, 0), (0, 126)))
            h7 = _make_k3f()(u, ust, g1, b1, w2p)

    bcol = jnp.pad(batch, (0, NPAD - N_NODES),
                   constant_values=G_GRAPHS).reshape(NPAD, 1)
    pooled = _make_pool()(bcol, h7)
    return pooled[:, :2]


# speedup vs baseline: 1.5710x; 1.5710x over previous
import functools

import jax
import jax.numpy as jnp
from jax import lax
from jax.experimental import pallas as pl
from jax.experimental.pallas import tpu as pltpu
from jax.experimental.pallas import tpu_sc as plsc

N_NODES = 10000
NPAD = 10240
E_EDGES = 320000
EPAD = 323584
G_GRAPHS = 64
BLK = 1024
GRID = NPAD // BLK
KCH = 128
EPT = EPAD // 16
NCH = EPT // KCH
GRP = 8
CPT = EPT // KCH
RPT = NPAD // 16


@functools.cache
def _make_agg(dh, split0):
    mesh = plsc.VectorSubcoreMesh(core_axis_name="c", subcore_axis_name="s")
    ept = (EPAD // 2 if split0 else EPAD) // 16

    @functools.partial(
        pl.kernel,
        out_type=jax.ShapeDtypeStruct((2 * NPAD, dh), jnp.float32),
        mesh=mesh,
        scratch_types=[
            pltpu.VMEM((KCH,), jnp.int32),
            pltpu.VMEM((KCH,), jnp.int32),
            pltpu.VMEM((KCH, dh), jnp.float32),
            pltpu.VMEM_SHARED((NPAD, dh), jnp.float32),
            pltpu.SemaphoreType.DMA,
        ],
    )
    def agg(h_hbm, src_hbm, dst_hbm, out_hbm, idxs, idxd, rows, accum, sem):
        c = lax.axis_index("c")
        s = lax.axis_index("s")

        def _zrow(i, _):
            for g in range(dh // 16):
                rows[i, pl.ds(g * 16, 16)] = jnp.zeros((16,), jnp.float32)
            return 0

        lax.fori_loop(0, KCH, _zrow, 0)

        def _zacc(k, _):
            pltpu.sync_copy(rows, accum.at[pl.ds(s * RPT + k * KCH, KCH)])
            return 0

        lax.fori_loop(0, RPT // KCH, _zacc, 0)
        plsc.subcore_barrier()

        def _step(i, _):
            eb = c * (16 * ept) + s * ept + i * KCH if split0 \
                else s * ept + i * KCH
            sb = eb if split0 else c * EPAD + eb
            pltpu.sync_copy(src_hbm.at[pl.ds(sb, KCH)], idxs)
            pltpu.sync_copy(dst_hbm.at[pl.ds(eb, KCH)], idxd)
            pltpu.async_copy(h_hbm.at[idxs], rows, sem).wait()
            pltpu.sync_copy(rows, accum.at[idxd], add=True)
            return 0

        lax.fori_loop(0, ept // KCH, _step, 0)
        plsc.subcore_barrier()

        pltpu.sync_copy(accum.at[pl.ds(s * RPT, RPT)],
                        out_hbm.at[pl.ds(c * NPAD + s * RPT, RPT)])

    return agg


def _rowmask(j):
    r = lax.broadcasted_iota(jnp.int32, (BLK, 1), 0) + j * BLK
    return r < N_NODES


def _k2_body(dh, eps_ref, h0, h1, a0, a1, w1, u_ref, st_ref):
    j = pl.program_id(0)
    e = 1.0 + eps_ref[0, 0]
    x0 = e * h0[...] + a0[...]
    x1 = e * h1[...] + a1[...]
    dn = (((1,), (0,)), ((), ()))
    u = lax.dot_general(x0, w1[:dh, :], dn, preferred_element_type=jnp.float32)
    u = u + lax.dot_general(x1, w1[dh:, :], dn,
                            preferred_element_type=jnp.float32)
    u_ref[...] = u
    um = jnp.where(_rowmask(j), u, 0.0)
    s0 = jnp.sum(um, axis=0, keepdims=True)
    s1 = jnp.sum(um * um, axis=0, keepdims=True)
    upd = jnp.concatenate(
        [s0, s1, jnp.zeros((6, s0.shape[1]), jnp.float32)], axis=0)

    @pl.when(j == 0)
    def _():
        st_ref[...] = upd

    @pl.when(j > 0)
    def _():
        st_ref[...] = st_ref[...] + upd


@functools.cache
def _make_k2(dh):
    fullspec = lambda shp: pl.BlockSpec(shp, lambda j: (0, 0))
    half = lambda off: pl.BlockSpec((BLK, dh), lambda j, o=off: (o + j, 0))
    return pl.pallas_call(
        functools.partial(_k2_body, dh),
        grid=(GRID,),
        in_specs=[
            pl.BlockSpec(memory_space=pltpu.SMEM),
            half(0), half(GRID),
            half(0), half(GRID),
            fullspec((2 * dh, 128)),
        ],
        out_specs=[
            pl.BlockSpec((BLK, 128), lambda j: (j, 0)),
            fullspec((8, 128)),
        ],
        out_shape=[
            jax.ShapeDtypeStruct((NPAD, 128), jnp.float32),
            jax.ShapeDtypeStruct((8, 128), jnp.float32),
        ],
    )


def _bn_coeffs(st_ref, g_ref, b_ref):
    mu = st_ref[0:1, :] / N_NODES
    var = st_ref[1:2, :] / N_NODES - mu * mu
    sc = g_ref[...] * lax.rsqrt(var + 1e-5)
    sh = b_ref[...] - mu * sc
    return sc, sh


def _k3_body(u, st, g1, b1, w2, v_ref, vst_ref):
    j = pl.program_id(0)
    sc, sh = _bn_coeffs(st, g1, b1)
    h2 = jnp.maximum(u[...] * sc + sh, 0.0)
    v = lax.dot_general(h2, w2[...], (((1,), (0,)), ((), ())),
                        preferred_element_type=jnp.float32)
    v_ref[...] = v
    vm = jnp.where(_rowmask(j), v, 0.0)
    s0 = jnp.sum(vm, axis=0, keepdims=True)
    s1 = jnp.sum(vm * vm, axis=0, keepdims=True)
    upd = jnp.concatenate(
        [s0, s1, jnp.zeros((6, s0.shape[1]), jnp.float32)], axis=0)

    @pl.when(j == 0)
    def _():
        vst_ref[...] = upd

    @pl.when(j > 0)
    def _():
        vst_ref[...] = vst_ref[...] + upd


_K3 = None


def _make_k3():
    global _K3
    if _K3 is None:
        fullspec = lambda shp: pl.BlockSpec(shp, lambda j: (0, 0))
        _K3 = pl.pallas_call(
            _k3_body,
            grid=(GRID,),
            in_specs=[
                pl.BlockSpec((BLK, 128), lambda j: (j, 0)),
                fullspec((8, 128)),
                fullspec((1, 128)), fullspec((1, 128)),
                fullspec((128, 256)),
            ],
            out_specs=[
                pl.BlockSpec((BLK, 256), lambda j: (j, 0)),
                fullspec((8, 256)),
            ],
            out_shape=[
                jax.ShapeDtypeStruct((NPAD, 256), jnp.float32),
                jax.ShapeDtypeStruct((8, 256), jnp.float32),
            ],
        )
    return _K3


def _k3f_body(u, st, g1, b1, w2, v_ref):
    sc, sh = _bn_coeffs(st, g1, b1)
    h2 = jnp.maximum(u[...] * sc + sh, 0.0)
    v = lax.dot_general(h2, w2[...], (((1,), (0,)), ((), ())),
                        preferred_element_type=jnp.float32)
    v_ref[...] = jnp.maximum(v, 0.0)


_K3F = None


def _make_k3f():
    global _K3F
    if _K3F is None:
        fullspec = lambda shp: pl.BlockSpec(shp, lambda j: (0, 0))
        _K3F = pl.pallas_call(
            _k3f_body,
            grid=(GRID,),
            in_specs=[
                pl.BlockSpec((BLK, 128), lambda j: (j, 0)),
                fullspec((8, 128)),
                fullspec((1, 128)), fullspec((1, 128)),
                fullspec((128, 128)),
            ],
            out_specs=pl.BlockSpec((BLK, 128), lambda j: (j, 0)),
            out_shape=jax.ShapeDtypeStruct((NPAD, 128), jnp.float32),
        )
    return _K3F


def _k1_body(v, vst, g2, b2, h_ref):
    sc, sh = _bn_coeffs(vst, g2, b2)
    h_ref[...] = jnp.maximum(v[...] * sc + sh, 0.0)


_K1 = None


def _make_k1():
    global _K1
    if _K1 is None:
        _K1 = pl.pallas_call(
            _k1_body,
            grid=(2, GRID),
            in_specs=[
                pl.BlockSpec((BLK, 128), lambda c, j: (j, c)),
                pl.BlockSpec((8, 128), lambda c, j: (0, c)),
                pl.BlockSpec((1, 128), lambda c, j: (0, c)),
                pl.BlockSpec((1, 128), lambda c, j: (0, c)),
            ],
            out_specs=pl.BlockSpec((BLK, 128), lambda c, j: (c * GRID + j, 0)),
            out_shape=jax.ShapeDtypeStruct((2 * NPAD, 128), jnp.float32),
        )
    return _K1


def _pool_body(bcol, h, out_ref, acc, cnt):
    j = pl.program_id(0)
    gids = lax.broadcasted_iota(jnp.int32, (BLK, G_GRAPHS), 1)
    oh = (bcol[...] == gids).astype(jnp.float32)
    dn = (((0,), (0,)), ((), ()))

    @pl.when(j == 0)
    def _():
        acc[...] = jnp.zeros((G_GRAPHS, 128), jnp.float32)
        cnt[...] = jnp.zeros((G_GRAPHS, 128), jnp.float32)

    acc[...] = acc[...] + lax.dot_general(
        oh, h[...], dn, preferred_element_type=jnp.float32)
    cnt[...] = cnt[...] + lax.dot_general(
        oh, jnp.ones((BLK, 128), jnp.float32), dn,
        preferred_element_type=jnp.float32)
    out_ref[...] = acc[...] / jnp.maximum(cnt[...], 1.0)


_POOL = None


def _make_pool():
    global _POOL
    if _POOL is None:
        _POOL = pl.pallas_call(
            _pool_body,
            grid=(GRID,),
            in_specs=[
                pl.BlockSpec((BLK, 1), lambda j: (j, 0)),
                pl.BlockSpec((BLK, 128), lambda j: (j, 0)),
            ],
            out_specs=pl.BlockSpec((G_GRAPHS, 128), lambda j: (0, 0)),
            out_shape=jax.ShapeDtypeStruct((G_GRAPHS, 128), jnp.float32),
            scratch_shapes=[
                pltpu.VMEM((G_GRAPHS, 128), jnp.float32),
                pltpu.VMEM((G_GRAPHS, 128), jnp.float32),
            ],
        )
    return _POOL


def kernel(x, edge_index, edge_attribute, batch, params):
    src = edge_index[0]
    dst = edge_index[1]
    npad = EPAD - E_EDGES
    srcp = jnp.concatenate([src, jnp.zeros((npad,), jnp.int32)])
    dstp = jnp.concatenate([dst, jnp.full((npad,), N_NODES, jnp.int32)])
    src2 = jnp.concatenate([srcp, srcp + NPAD])

    xp = jnp.pad(x, ((0, NPAD - N_NODES), (0, 0)))
    hcat = jnp.concatenate([xp, jnp.zeros((NPAD, 128), jnp.float32)], axis=0)

    h7 = None
    for i, p in enumerate(params):
        if i == 0:
            agg = _make_agg(128, True)(hcat, srcp, dstp)
            w1 = jnp.concatenate([p['W1'], p['W1']], axis=0)
        else:
            agg = _make_agg(128, False)(hcat, src2, dstp)
            w1 = p['W1']
        eps2 = (p['eps'].astype(jnp.float32)).reshape(1, 1)
        u, ust = _make_k2(128)(eps2, hcat, hcat, agg, agg, w1)
        g1 = p['g1'].reshape(1, 128)
        b1 = p['b1'].reshape(1, 128)
        if i < 6:
            v, vst = _make_k3()(u, ust, g1, b1, p['W2'])
            hcat = _make_k1()(v, vst, p['g2'].reshape(1, 256),
                              p['b2'].reshape(1, 256))
        else:
            w2p = jnp.pad(p['W2'], ((0, 0), (0, 126)))
            h7 = _make_k3f()(u, ust, g1, b1, w2p)

    bcol = jnp.pad(batch, (0, NPAD - N_NODES),
                   constant_values=G_GRAPHS).reshape(NPAD, 1)
    pooled = _make_pool()(bcol, h7)
    return pooled[:, :2]


# speedup vs baseline: 1.9226x; 1.2238x over previous
import functools

import jax
import jax.numpy as jnp
from jax import lax
from jax.experimental import pallas as pl
from jax.experimental.pallas import tpu as pltpu
from jax.experimental.pallas import tpu_sc as plsc

N_NODES = 10000
NPAD = 10240
E_EDGES = 320000
EPAD = 323584
G_GRAPHS = 64
BLK = 1024
GRID = NPAD // BLK
KCH = 128
EPT = EPAD // 16
NCH = EPT // KCH
RPT = NPAD // 16


@functools.cache
def _make_agg(dh, split0):
    mesh = plsc.VectorSubcoreMesh(core_axis_name="c", subcore_axis_name="s")
    ncht = (EPAD // 2 if split0 else EPAD) // KCH // 16

    @functools.partial(
        pl.kernel,
        out_type=jax.ShapeDtypeStruct((2 * NPAD, dh), jnp.float32),
        mesh=mesh,
        scratch_types=[
            pltpu.VMEM((KCH,), jnp.int32),
            pltpu.VMEM((KCH,), jnp.int32),
            pltpu.VMEM((KCH,), jnp.int32),
            pltpu.VMEM((KCH,), jnp.int32),
            pltpu.VMEM((KCH, dh), jnp.float32),
            pltpu.VMEM((KCH, dh), jnp.float32),
            pltpu.VMEM_SHARED((NPAD, dh), jnp.float32),
            pltpu.SemaphoreType.DMA,
            pltpu.SemaphoreType.DMA,
        ],
    )
    def agg(h_hbm, src_hbm, dst_hbm, out_hbm, idxsa, idxda, idxsb, idxdb,
            rowsa, rowsb, accum, sema, semb):
        c = lax.axis_index("c")
        s = lax.axis_index("s")

        def _zrow(i, _):
            for g in range(dh // 16):
                rowsa[i, pl.ds(g * 16, 16)] = jnp.zeros((16,), jnp.float32)
            return 0

        lax.fori_loop(0, KCH, _zrow, 0)

        def _zacc(k, _):
            pltpu.sync_copy(rowsa, accum.at[pl.ds(s * RPT + k * KCH, KCH)])
            return 0

        lax.fori_loop(0, RPT // KCH, _zacc, 0)
        plsc.subcore_barrier()

        ebase = c * (EPAD // 2) if split0 else 0
        sbase = ebase if split0 else c * EPAD

        def _pair(i, _):
            ka = ebase + ((2 * i) * 16 + s) * KCH
            kb = ebase + ((2 * i + 1) * 16 + s) * KCH
            pltpu.sync_copy(src_hbm.at[pl.ds(sbase - ebase + ka, KCH)], idxsa)
            pltpu.sync_copy(dst_hbm.at[pl.ds(ka, KCH)], idxda)
            da = pltpu.async_copy(h_hbm.at[idxsa], rowsa, sema)
            pltpu.sync_copy(src_hbm.at[pl.ds(sbase - ebase + kb, KCH)], idxsb)
            pltpu.sync_copy(dst_hbm.at[pl.ds(kb, KCH)], idxdb)
            db = pltpu.async_copy(h_hbm.at[idxsb], rowsb, semb)
            da.wait()
            pltpu.sync_copy(rowsa, accum.at[idxda], add=True)
            db.wait()
            pltpu.sync_copy(rowsb, accum.at[idxdb], add=True)
            return 0

        def _step(i, _):
            ka = ebase + (i * 16 + s) * KCH
            pltpu.sync_copy(src_hbm.at[pl.ds(sbase - ebase + ka, KCH)], idxsa)
            pltpu.sync_copy(dst_hbm.at[pl.ds(ka, KCH)], idxda)
            pltpu.async_copy(h_hbm.at[idxsa], rowsa, sema).wait()
            pltpu.sync_copy(rowsa, accum.at[idxda], add=True)
            return 0

        if ncht % 2 == 0:
            lax.fori_loop(0, ncht // 2, _pair, 0)
        else:
            lax.fori_loop(0, ncht, _step, 0)
        plsc.subcore_barrier()

        pltpu.sync_copy(accum.at[pl.ds(s * RPT, RPT)],
                        out_hbm.at[pl.ds(c * NPAD + s * RPT, RPT)])

    return agg


def _rowmask(j):
    r = lax.broadcasted_iota(jnp.int32, (BLK, 1), 0) + j * BLK
    return r < N_NODES


def _dot(a, b, contract=(1, 0)):
    return lax.dot_general(a, b, (((contract[0],), (contract[1],)), ((), ())),
                           preferred_element_type=jnp.float32,
                           precision=lax.Precision.HIGHEST)


def _k2_body(dh, eps_ref, h0, h1, a0, a1, w1, u_ref, st_ref):
    j = pl.program_id(0)
    e = 1.0 + eps_ref[0, 0]
    x0 = e * h0[...] + a0[...]
    x1 = e * h1[...] + a1[...]
    dn = (((1,), (0,)), ((), ()))
    u = _dot(x0, w1[:dh, :]) + _dot(x1, w1[dh:, :])
    u_ref[...] = u
    um = jnp.where(_rowmask(j), u, 0.0)
    s0 = jnp.sum(um, axis=0, keepdims=True)
    s1 = jnp.sum(um * um, axis=0, keepdims=True)
    upd = jnp.concatenate(
        [s0, s1, jnp.zeros((6, s0.shape[1]), jnp.float32)], axis=0)

    @pl.when(j == 0)
    def _():
        st_ref[...] = upd

    @pl.when(j > 0)
    def _():
        st_ref[...] = st_ref[...] + upd


@functools.cache
def _make_k2(dh):
    fullspec = lambda shp: pl.BlockSpec(shp, lambda j: (0, 0))
    half = lambda off: pl.BlockSpec((BLK, dh), lambda j, o=off: (o + j, 0))
    return pl.pallas_call(
        functools.partial(_k2_body, dh),
        grid=(GRID,),
        in_specs=[
            pl.BlockSpec(memory_space=pltpu.SMEM),
            half(0), half(GRID),
            half(0), half(GRID),
            fullspec((2 * dh, 128)),
        ],
        out_specs=[
            pl.BlockSpec((BLK, 128), lambda j: (j, 0)),
            fullspec((8, 128)),
        ],
        out_shape=[
            jax.ShapeDtypeStruct((NPAD, 128), jnp.float32),
            jax.ShapeDtypeStruct((8, 128), jnp.float32),
        ],
    )


def _bn_coeffs(st_ref, g_ref, b_ref):
    mu = st_ref[0:1, :] / N_NODES
    var = st_ref[1:2, :] / N_NODES - mu * mu
    sc = g_ref[...] / jnp.sqrt(var + 1e-5)
    sh = b_ref[...] - mu * sc
    return sc, sh


def _k3_body(u, st, g1, b1, w2, v_ref, vst_ref):
    j = pl.program_id(0)
    sc, sh = _bn_coeffs(st, g1, b1)
    h2 = jnp.maximum(u[...] * sc + sh, 0.0)
    v = _dot(h2, w2[...])
    v_ref[...] = v
    vm = jnp.where(_rowmask(j), v, 0.0)
    s0 = jnp.sum(vm, axis=0, keepdims=True)
    s1 = jnp.sum(vm * vm, axis=0, keepdims=True)
    upd = jnp.concatenate(
        [s0, s1, jnp.zeros((6, s0.shape[1]), jnp.float32)], axis=0)

    @pl.when(j == 0)
    def _():
        vst_ref[...] = upd

    @pl.when(j > 0)
    def _():
        vst_ref[...] = vst_ref[...] + upd


_K3 = None


def _make_k3():
    global _K3
    if _K3 is None:
        fullspec = lambda shp: pl.BlockSpec(shp, lambda j: (0, 0))
        _K3 = pl.pallas_call(
            _k3_body,
            grid=(GRID,),
            in_specs=[
                pl.BlockSpec((BLK, 128), lambda j: (j, 0)),
                fullspec((8, 128)),
                fullspec((1, 128)), fullspec((1, 128)),
                fullspec((128, 256)),
            ],
            out_specs=[
                pl.BlockSpec((BLK, 256), lambda j: (j, 0)),
                fullspec((8, 256)),
            ],
            out_shape=[
                jax.ShapeDtypeStruct((NPAD, 256), jnp.float32),
                jax.ShapeDtypeStruct((8, 256), jnp.float32),
            ],
        )
    return _K3


def _k3f_body(u, st, g1, b1, w2, v_ref):
    sc, sh = _bn_coeffs(st, g1, b1)
    h2 = jnp.maximum(u[...] * sc + sh, 0.0)
    v = _dot(h2, w2[...])
    v_ref[...] = jnp.maximum(v, 0.0)


_K3F = None


def _make_k3f():
    global _K3F
    if _K3F is None:
        fullspec = lambda shp: pl.BlockSpec(shp, lambda j: (0, 0))
        _K3F = pl.pallas_call(
            _k3f_body,
            grid=(GRID,),
            in_specs=[
                pl.BlockSpec((BLK, 128), lambda j: (j, 0)),
                fullspec((8, 128)),
                fullspec((1, 128)), fullspec((1, 128)),
                fullspec((128, 128)),
            ],
            out_specs=pl.BlockSpec((BLK, 128), lambda j: (j, 0)),
            out_shape=jax.ShapeDtypeStruct((NPAD, 128), jnp.float32),
        )
    return _K3F


def _k1_body(v, vst, g2, b2, h_ref):
    sc, sh = _bn_coeffs(vst, g2, b2)
    h_ref[...] = jnp.maximum(v[...] * sc + sh, 0.0)


_K1 = None


def _make_k1():
    global _K1
    if _K1 is None:
        _K1 = pl.pallas_call(
            _k1_body,
            grid=(2, GRID),
            in_specs=[
                pl.BlockSpec((BLK, 128), lambda c, j: (j, c)),
                pl.BlockSpec((8, 128), lambda c, j: (0, c)),
                pl.BlockSpec((1, 128), lambda c, j: (0, c)),
                pl.BlockSpec((1, 128), lambda c, j: (0, c)),
            ],
            out_specs=pl.BlockSpec((BLK, 128), lambda c, j: (c * GRID + j, 0)),
            out_shape=jax.ShapeDtypeStruct((2 * NPAD, 128), jnp.float32),
        )
    return _K1


def _pool_body(bcol, h, out_ref, acc, cnt):
    j = pl.program_id(0)
    gids = lax.broadcasted_iota(jnp.int32, (BLK, G_GRAPHS), 1)
    oh = (bcol[...] == gids).astype(jnp.float32)
    dn = (((0,), (0,)), ((), ()))

    @pl.when(j == 0)
    def _():
        acc[...] = jnp.zeros((G_GRAPHS, 128), jnp.float32)
        cnt[...] = jnp.zeros((G_GRAPHS, 128), jnp.float32)

    acc[...] = acc[...] + _dot(oh, h[...], contract=(0, 0))
    cnt[...] = cnt[...] + _dot(oh, jnp.ones((BLK, 128), jnp.float32),
                               contract=(0, 0))
    out_ref[...] = acc[...] / jnp.maximum(cnt[...], 1.0)


_POOL = None


def _make_pool():
    global _POOL
    if _POOL is None:
        _POOL = pl.pallas_call(
            _pool_body,
            grid=(GRID,),
            in_specs=[
                pl.BlockSpec((BLK, 1), lambda j: (j, 0)),
                pl.BlockSpec((BLK, 128), lambda j: (j, 0)),
            ],
            out_specs=pl.BlockSpec((G_GRAPHS, 128), lambda j: (0, 0)),
            out_shape=jax.ShapeDtypeStruct((G_GRAPHS, 128), jnp.float32),
            scratch_shapes=[
                pltpu.VMEM((G_GRAPHS, 128), jnp.float32),
                pltpu.VMEM((G_GRAPHS, 128), jnp.float32),
            ],
        )
    return _POOL


def kernel(x, edge_index, edge_attribute, batch, params):
    src = edge_index[0]
    dst = edge_index[1]
    npad = EPAD - E_EDGES
    srcp = jnp.concatenate([src, jnp.zeros((npad,), jnp.int32)])
    dstp = jnp.concatenate([dst, jnp.full((npad,), N_NODES, jnp.int32)])
    src2 = jnp.concatenate([srcp, srcp + NPAD])

    xp = jnp.pad(x, ((0, NPAD - N_NODES), (0, 0)))
    hcat = jnp.concatenate([xp, jnp.zeros((NPAD, 128), jnp.float32)], axis=0)

    h7 = None
    for i, p in enumerate(params):
        if i == 0:
            agg = _make_agg(128, True)(hcat, srcp, dstp)
            w1 = jnp.concatenate([p['W1'], p['W1']], axis=0)
        else:
            agg = _make_agg(128, False)(hcat, src2, dstp)
            w1 = p['W1']
        eps2 = (p['eps'].astype(jnp.float32)).reshape(1, 1)
        u, ust = _make_k2(128)(eps2, hcat, hcat, agg, agg, w1)
        g1 = p['g1'].reshape(1, 128)
        b1 = p['b1'].reshape(1, 128)
        if i < 6:
            v, vst = _make_k3()(u, ust, g1, b1, p['W2'])
            hcat = _make_k1()(v, vst, p['g2'].reshape(1, 256),
                              p['b2'].reshape(1, 256))
        else:
            w2p = jnp.pad(p['W2'], ((0, 0), (0, 126)))
            h7 = _make_k3f()(u, ust, g1, b1, w2p)

    bcol = jnp.pad(batch, (0, NPAD - N_NODES),
                   constant_values=G_GRAPHS).reshape(NPAD, 1)
    pooled = _make_pool()(bcol, h7)
    return pooled[:, :2]
